# Initial kernel scaffold; baseline (speedup 1.0000x reference)
import jax
import jax.numpy as jnp
from jax.experimental import pallas as pl


def kernel(h_t, batch):
    raise NotImplementedError("write your pallas kernel here")


# speedup vs baseline: 7.8245x; 7.8245x over previous
import functools

import jax
import jax.numpy as jnp
from jax import lax
from jax.experimental import pallas as pl
from jax.experimental.pallas import tpu as pltpu
from jax.experimental.pallas import tpu_sc as plsc

_NUM_SEGMENTS = 1024
_D = 128
_NC = 2
_NS = 16
_NW = _NC * _NS
_CHUNK = 128
_SEG_PER_TILE = _NUM_SEGMENTS // _NS


def _sc_segment_sum(h_t, idx_stream, zeros64, *, nch, max_chunks_per_worker):
  n = h_t.shape[0]

  def body(h_hbm, idx_hbm, z_hbm, out_hbm, acc, idx_v, rows_v):
    c = lax.axis_index("c")
    s = lax.axis_index("s")
    wid = c * _NS + s

    pltpu.sync_copy(z_hbm, acc.at[pl.ds(s * _SEG_PER_TILE, _SEG_PER_TILE)])

    @pl.when(s == 0)
    def _():
      pltpu.sync_copy(z_hbm.at[pl.ds(0, 1)], acc.at[pl.ds(_NUM_SEGMENTS, 1)])

    lo = wid * nch // _NW
    hi = (wid + 1) * nch // _NW
    pltpu.sync_copy(idx_hbm.at[wid], idx_v)
    plsc.subcore_barrier()

    def step(j, carry):
      @pl.when(j < hi - lo)
      def _():
        r = lo + j
        off = lax.min(r * _CHUNK, n - _CHUNK)
        pltpu.sync_copy(h_hbm.at[pl.ds(off, _CHUNK)], rows_v)
        pltpu.sync_copy(rows_v, acc.at[idx_v.at[j]], add=True)
      return carry

    lax.fori_loop(0, max_chunks_per_worker, step, 0)
    plsc.subcore_barrier()

    pltpu.sync_copy(
        acc.at[pl.ds(s * _SEG_PER_TILE, _SEG_PER_TILE)],
        out_hbm.at[c, pl.ds(s * _SEG_PER_TILE, _SEG_PER_TILE)])

  mesh = plsc.VectorSubcoreMesh(core_axis_name="c", subcore_axis_name="s")
  run = pl.kernel(
      body,
      out_type=jax.ShapeDtypeStruct((_NC, _NUM_SEGMENTS, _D), jnp.float32),
      mesh=mesh,
      scratch_types=[
          pltpu.VMEM_SHARED((_NUM_SEGMENTS + 1, _D), jnp.float32),
          pltpu.VMEM((max_chunks_per_worker, _CHUNK), jnp.int32),
          pltpu.VMEM((_CHUNK, _D), jnp.float32),
      ],
  )
  return run(h_t, idx_stream, zeros64)


def _merge_body(p_ref, o_ref):
  o_ref[...] = p_ref[0] + p_ref[1]


@jax.jit
def kernel(h_t, batch):
  n, d = h_t.shape
  nch = -(-n // _CHUNK)
  dup = nch * _CHUNK - n

  idx32 = batch.astype(jnp.int32)
  head = idx32[: (nch - 1) * _CHUNK]
  tail = idx32[(nch - 1) * _CHUNK :]
  dump = jnp.full((dup,), _NUM_SEGMENTS, jnp.int32)
  idx_stream = jnp.concatenate([head, dump, tail]).reshape(nch, _CHUNK)

  zeros64 = jnp.zeros((_SEG_PER_TILE, _D), jnp.float32)
  max_chunks = -(-nch // _NW)
  rows = (jnp.arange(_NW)[:, None] * nch // _NW) + jnp.arange(max_chunks)
  idx_stream = jnp.take(idx_stream, rows, axis=0)

  partials = _sc_segment_sum(
      h_t, idx_stream, zeros64, nch=nch, max_chunks_per_worker=max_chunks)

  return pl.pallas_call(
      _merge_body,
      out_shape=jax.ShapeDtypeStruct((_NUM_SEGMENTS, _D), jnp.float32),
  )(partials)
